# baseline (speedup 1.0000x reference)
import functools
import jax
import jax.numpy as jnp
from jax import lax
from jax.experimental import pallas as pl
from jax.experimental.pallas import tpu as pltpu
from jax.experimental.pallas import tpu_sc as plsc

NC = 2
NS = 16
LANES = 16

EPW = 128


def _sc_aggregate(E_pad, N, DH, TPT, NROWS_SP):
    NWIN = TPT // EPW
    ZROWS = NROWS_SP // NS
    WOUT = 640
    LAST = N - (NS - 1) * WOUT

    mesh = plsc.VectorSubcoreMesh(core_axis_name="c", subcore_axis_name="s")

    @functools.partial(
        pl.kernel,
        out_type=[
            jax.ShapeDtypeStruct((2, N, DH), jnp.float32),
            jax.ShapeDtypeStruct((2, N, DH), jnp.float32),
            jax.ShapeDtypeStruct((NROWS_SP,), jnp.float32),
            jax.ShapeDtypeStruct((NROWS_SP,), jnp.float32),
        ],
        mesh=mesh,
        scratch_types=[
            pltpu.VMEM((3, 2, EPW), jnp.int32),
            pltpu.VMEM((3, EPW, DH), jnp.float32),
            pltpu.VMEM((EPW,), jnp.float32),
            pltpu.MemorySpace.VMEM_SHARED((NROWS_SP, DH), jnp.float32),
            pltpu.MemorySpace.VMEM_SHARED((NROWS_SP,), jnp.float32),
            pltpu.SemaphoreType.DMA((3,)),
            pltpu.SemaphoreType.DMA((3,)),
            pltpu.SemaphoreType.DMA((3,)),
            pltpu.SemaphoreType.DMA((3,)),
        ],
    )
    def agg(i_hbm, xr_hbm, z2d_hbm, z1d_hbm, sin_h, sout_h, cin, cout,
            ibuf, rows, ones, acc_sp, cnt_sp, si, sg, ss, sc):
        c = lax.axis_index("c")
        s = lax.axis_index("s")

        def fill_ones(i, _):
            ones[pl.ds(i * LANES, LANES)] = jnp.ones((LANES,), jnp.float32)
            return 0
        lax.fori_loop(0, EPW // LANES, fill_ones, 0)

        for r in range(2):
            xsrc = xr_hbm
            wbase = (c * NS + s) * NWIN

            pltpu.make_async_copy(i_hbm.at[r, wbase], ibuf.at[0],
                                  si.at[0]).start()

            pltpu.sync_copy(z2d_hbm, acc_sp.at[pl.ds(s * ZROWS, ZROWS), :])
            if r == 0:
                @pl.when(s == 0)
                def _():
                    pltpu.sync_copy(z1d_hbm, cnt_sp)
            plsc.subcore_barrier()

            pltpu.make_async_copy(i_hbm.at[r, wbase + 1], ibuf.at[1],
                                  si.at[1]).start()
            pltpu.make_async_copy(i_hbm.at[r, wbase], ibuf.at[0],
                                  si.at[0]).wait()
            pltpu.async_copy(xsrc.at[ibuf.at[0, 0]], rows.at[0], sg.at[0])

            def step(w, _):
                s0 = lax.rem(w, 3)
                s1 = lax.rem(w + 1, 3)
                s2 = lax.rem(w + 2, 3)

                @pl.when(w + 1 < NWIN)
                def _():
                    pltpu.make_async_copy(i_hbm.at[r, wbase + w + 1],
                                          ibuf.at[s1], si.at[s1]).wait()
                    pltpu.async_copy(xsrc.at[ibuf.at[s1, 0]], rows.at[s1],
                                     sg.at[s1])

                pltpu.make_async_copy(xsrc.at[ibuf.at[s0, 0]], rows.at[s0],
                                      sg.at[s0]).wait()
                pltpu.async_copy(rows.at[s0], acc_sp.at[ibuf.at[s0, 1]],
                                 ss.at[s0], add=True)
                if r == 0:
                    pltpu.async_copy(ones, cnt_sp.at[ibuf.at[s0, 1]],
                                     sc.at[s0], add=True)

                @pl.when(w + 2 < NWIN)
                def _():
                    @pl.when(w >= 1)
                    def _():
                        pltpu.make_async_copy(rows.at[s2],
                                              acc_sp.at[ibuf.at[s2, 1]],
                                              ss.at[s2]).wait()
                        if r == 0:
                            pltpu.make_async_copy(ones,
                                                  cnt_sp.at[ibuf.at[s2, 1]],
                                                  sc.at[s2]).wait()
                    pltpu.make_async_copy(i_hbm.at[r, wbase + w + 2],
                                          ibuf.at[s2], si.at[s2]).start()
                return 0
            lax.fori_loop(0, NWIN, step, 0)

            for wl in range(NWIN - 3, NWIN):
                pq = wl % 3
                pltpu.make_async_copy(rows.at[pq], acc_sp.at[ibuf.at[pq, 1]],
                                      ss.at[pq]).wait()
                if r == 0:
                    pltpu.make_async_copy(ones, cnt_sp.at[ibuf.at[pq, 1]],
                                          sc.at[pq]).wait()
            plsc.subcore_barrier()

            for ci_, dst in ((0, sin_h), (1, sout_h)):
                @pl.when((c == ci_) & (s < NS - 1))
                def _(dst=dst):
                    pltpu.sync_copy(acc_sp.at[pl.ds(s * WOUT, WOUT), :],
                                    dst.at[r, pl.ds(s * WOUT, WOUT), :])

                @pl.when((c == ci_) & (s == NS - 1))
                def _(dst=dst):
                    pltpu.sync_copy(acc_sp.at[pl.ds((NS - 1) * WOUT, LAST), :],
                                    dst.at[r, pl.ds((NS - 1) * WOUT, LAST), :])

            if r == 0:
                @pl.when((c == 0) & (s == 0))
                def _():
                    pltpu.sync_copy(cnt_sp, cin)

                @pl.when((c == 1) & (s == 0))
                def _():
                    pltpu.sync_copy(cnt_sp, cout)

            plsc.subcore_barrier()

    return agg


def _merge_body(ws_ref, wi_ref, wo_ref, wc_ref, bs_ref, bi_ref, bo_ref, bc_ref,
                w1_ref, w2_ref, w3_ref, bb_ref, bbi_ref, bbo_ref):
    D = ws_ref.shape[0]
    wc1 = wc_ref[0:D, :]
    wc2 = wc_ref[D:2 * D, :]
    wc3 = wc_ref[2 * D:3 * D, :]
    w1_ref[...] = jnp.dot(ws_ref[...], wc1, preferred_element_type=jnp.float32)
    w2_ref[...] = jnp.dot(wi_ref[...], wc2, preferred_element_type=jnp.float32)
    w3_ref[...] = jnp.dot(wo_ref[...], wc3, preferred_element_type=jnp.float32)
    bb_ref[...] = jnp.dot(bs_ref[...], wc1, preferred_element_type=jnp.float32) + bc_ref[...]
    bbi_ref[...] = jnp.dot(bi_ref[...], wc2, preferred_element_type=jnp.float32)
    bbo_ref[...] = jnp.dot(bo_ref[...], wc3, preferred_element_type=jnp.float32)


def _final_body(x_ref, sin_ref, sout_ref, cin_ref, cout_ref,
                w1_ref, w2_ref, w3_ref, bb_ref, bbi_ref, bbo_ref, out_ref):
    ci = cin_ref[...]
    co = cout_ref[...]
    inv_i = 1.0 / jnp.maximum(ci, 1.0)
    inv_o = 1.0 / jnp.maximum(co, 1.0)
    ain = jnp.concatenate([sin_ref[0], sin_ref[1]], axis=-1) * inv_i
    aout = jnp.concatenate([sout_ref[0], sout_ref[1]], axis=-1) * inv_o
    acc = jnp.dot(x_ref[...], w1_ref[...], preferred_element_type=jnp.float32)
    acc += jnp.dot(ain, w2_ref[...], preferred_element_type=jnp.float32)
    acc += jnp.dot(aout, w3_ref[...], preferred_element_type=jnp.float32)
    acc += bb_ref[...]
    acc += jnp.where(ci > 0.0, 1.0, 0.0) * bbi_ref[...]
    acc += jnp.where(co > 0.0, 1.0, 0.0) * bbo_ref[...]
    out_ref[...] = jnp.maximum(acc, 0.0)


def kernel(x, senders, receivers, W_self, b_self, W_in, b_in, W_out, b_out,
           W_comb, b_comb):
    N, D = x.shape
    E = senders.shape[0]
    DH = D // 2

    TPT = -(-E // NS // EPW) * EPW
    E_pad = NS * TPT
    PAD = E_pad - E
    NROWS_SP = -(-(N + 1) // NS) * NS

    g_pad = jnp.arange(PAD, dtype=jnp.int32) % N
    s_pad = jnp.full((PAD,), N, dtype=jnp.int32)
    g_flat = jnp.concatenate([senders, g_pad, receivers, g_pad])
    s_flat = jnp.concatenate([receivers, s_pad, senders, s_pad])
    g2 = (2 * g_flat).reshape(-1, EPW)
    s2 = s_flat.reshape(-1, EPW)
    i_all = jnp.stack([jnp.stack([g2, s2], axis=1),
                       jnp.stack([g2 + 1, s2], axis=1)])
    xr = x.reshape(2 * N, DH)
    z2d = jnp.zeros((NROWS_SP // NS, DH), jnp.float32)
    z1d = jnp.zeros((NROWS_SP,), jnp.float32)

    agg = _sc_aggregate(E_pad, N, DH, TPT, NROWS_SP)
    sin_h, sout_h, cin_pad, cout_pad = agg(i_all, xr, z2d, z1d)
    cin = cin_pad[:N]
    cout = cout_pad[:N]

    w1, w2, w3, bb, bbi, bbo = pl.pallas_call(
        _merge_body,
        out_shape=[
            jax.ShapeDtypeStruct((D, D), jnp.float32),
            jax.ShapeDtypeStruct((D, D), jnp.float32),
            jax.ShapeDtypeStruct((D, D), jnp.float32),
            jax.ShapeDtypeStruct((1, D), jnp.float32),
            jax.ShapeDtypeStruct((1, D), jnp.float32),
            jax.ShapeDtypeStruct((1, D), jnp.float32),
        ],
    )(W_self, W_in, W_out, W_comb,
      b_self.reshape(1, D), b_in.reshape(1, D), b_out.reshape(1, D),
      b_comb.reshape(1, D))

    BM = 2000
    grid = N // BM
    out = pl.pallas_call(
        _final_body,
        grid=(grid,),
        in_specs=[
            pl.BlockSpec((BM, D), lambda i: (i, 0)),
            pl.BlockSpec((2, BM, DH), lambda i: (0, i, 0)),
            pl.BlockSpec((2, BM, DH), lambda i: (0, i, 0)),
            pl.BlockSpec((BM, 1), lambda i: (i, 0)),
            pl.BlockSpec((BM, 1), lambda i: (i, 0)),
            pl.BlockSpec((D, D), lambda i: (0, 0)),
            pl.BlockSpec((D, D), lambda i: (0, 0)),
            pl.BlockSpec((D, D), lambda i: (0, 0)),
            pl.BlockSpec((1, D), lambda i: (0, 0)),
            pl.BlockSpec((1, D), lambda i: (0, 0)),
            pl.BlockSpec((1, D), lambda i: (0, 0)),
        ],
        out_specs=pl.BlockSpec((BM, D), lambda i: (i, 0)),
        out_shape=jax.ShapeDtypeStruct((N, D), jnp.float32),
    )(x, sin_h, sout_h, cin.reshape(N, 1), cout.reshape(N, 1),
      w1, w2, w3, bb, bbi, bbo)
    return out


# speedup vs baseline: 1.0379x; 1.0379x over previous
import functools
import jax
import jax.numpy as jnp
from jax import lax
from jax.experimental import pallas as pl
from jax.experimental.pallas import tpu as pltpu
from jax.experimental.pallas import tpu_sc as plsc

NC = 2
NS = 16
LANES = 16

EPW = 128


def _sc_aggregate(E_pad, N, DH, TPT, NROWS_SP):
    NWIN = TPT // EPW
    ZROWS = NROWS_SP // NS
    WOUT = 640
    LAST = N - (NS - 1) * WOUT

    mesh = plsc.VectorSubcoreMesh(core_axis_name="c", subcore_axis_name="s")

    @functools.partial(
        pl.kernel,
        out_type=[
            jax.ShapeDtypeStruct((2, N, DH), jnp.float32),
            jax.ShapeDtypeStruct((2, N, DH), jnp.float32),
            jax.ShapeDtypeStruct((NROWS_SP,), jnp.float32),
            jax.ShapeDtypeStruct((NROWS_SP,), jnp.float32),
        ],
        mesh=mesh,
        scratch_types=[
            pltpu.VMEM((3, 2, EPW), jnp.int32),
            pltpu.VMEM((3, EPW, DH), jnp.float32),
            pltpu.VMEM((EPW,), jnp.float32),
            pltpu.MemorySpace.VMEM_SHARED((NROWS_SP, DH), jnp.float32),
            pltpu.MemorySpace.VMEM_SHARED((NROWS_SP,), jnp.float32),
            pltpu.SemaphoreType.DMA((3,)),
            pltpu.SemaphoreType.DMA((3,)),
            pltpu.SemaphoreType.DMA((3,)),
            pltpu.SemaphoreType.DMA((3,)),
        ],
    )
    def agg(i_hbm, x0_hbm, x1_hbm, z2d_hbm, z1d_hbm, sin_h, sout_h, cin, cout,
            ibuf, rows, ones, acc_sp, cnt_sp, si, sg, ss, sc):
        c = lax.axis_index("c")
        s = lax.axis_index("s")

        def fill_ones(i, _):
            ones[pl.ds(i * LANES, LANES)] = jnp.ones((LANES,), jnp.float32)
            return 0
        lax.fori_loop(0, EPW // LANES, fill_ones, 0)

        for r in range(2):
            xsrc = x0_hbm if r == 0 else x1_hbm
            wbase = (c * NS + s) * NWIN

            pltpu.make_async_copy(i_hbm.at[wbase], ibuf.at[0],
                                  si.at[0]).start()

            pltpu.sync_copy(z2d_hbm, acc_sp.at[pl.ds(s * ZROWS, ZROWS), :])
            if r == 0:
                @pl.when(s == 0)
                def _():
                    pltpu.sync_copy(z1d_hbm, cnt_sp)
            plsc.subcore_barrier()

            pltpu.make_async_copy(i_hbm.at[wbase + 1], ibuf.at[1],
                                  si.at[1]).start()
            pltpu.make_async_copy(i_hbm.at[wbase], ibuf.at[0],
                                  si.at[0]).wait()
            pltpu.async_copy(xsrc.at[ibuf.at[0, 0]], rows.at[0], sg.at[0])

            def step(w, _):
                s0 = lax.rem(w, 3)
                s1 = lax.rem(w + 1, 3)
                s2 = lax.rem(w + 2, 3)

                @pl.when(w + 1 < NWIN)
                def _():
                    pltpu.make_async_copy(i_hbm.at[wbase + w + 1], ibuf.at[s1],
                                          si.at[s1]).wait()
                    pltpu.async_copy(xsrc.at[ibuf.at[s1, 0]], rows.at[s1],
                                     sg.at[s1])

                pltpu.make_async_copy(xsrc.at[ibuf.at[s0, 0]], rows.at[s0],
                                      sg.at[s0]).wait()
                pltpu.async_copy(rows.at[s0], acc_sp.at[ibuf.at[s0, 1]],
                                 ss.at[s0], add=True)
                if r == 0:
                    pltpu.async_copy(ones, cnt_sp.at[ibuf.at[s0, 1]],
                                     sc.at[s0], add=True)

                @pl.when(w + 2 < NWIN)
                def _():
                    @pl.when(w >= 1)
                    def _():
                        pltpu.make_async_copy(rows.at[s2],
                                              acc_sp.at[ibuf.at[s2, 1]],
                                              ss.at[s2]).wait()
                        if r == 0:
                            pltpu.make_async_copy(ones,
                                                  cnt_sp.at[ibuf.at[s2, 1]],
                                                  sc.at[s2]).wait()
                    pltpu.make_async_copy(i_hbm.at[wbase + w + 2], ibuf.at[s2],
                                          si.at[s2]).start()
                return 0
            lax.fori_loop(0, NWIN, step, 0)

            for wl in range(NWIN - 3, NWIN):
                pq = wl % 3
                pltpu.make_async_copy(rows.at[pq], acc_sp.at[ibuf.at[pq, 1]],
                                      ss.at[pq]).wait()
                if r == 0:
                    pltpu.make_async_copy(ones, cnt_sp.at[ibuf.at[pq, 1]],
                                          sc.at[pq]).wait()
            plsc.subcore_barrier()

            for ci_, dst in ((0, sin_h), (1, sout_h)):
                @pl.when((c == ci_) & (s < NS - 1))
                def _(dst=dst):
                    pltpu.sync_copy(acc_sp.at[pl.ds(s * WOUT, WOUT), :],
                                    dst.at[r, pl.ds(s * WOUT, WOUT), :])

                @pl.when((c == ci_) & (s == NS - 1))
                def _(dst=dst):
                    pltpu.sync_copy(acc_sp.at[pl.ds((NS - 1) * WOUT, LAST), :],
                                    dst.at[r, pl.ds((NS - 1) * WOUT, LAST), :])

            if r == 0:
                @pl.when((c == 0) & (s == 0))
                def _():
                    pltpu.sync_copy(cnt_sp, cin)

                @pl.when((c == 1) & (s == 0))
                def _():
                    pltpu.sync_copy(cnt_sp, cout)

            plsc.subcore_barrier()

    return agg


def _merge_body(ws_ref, wi_ref, wo_ref, wc_ref, bs_ref, bi_ref, bo_ref, bc_ref,
                w1_ref, w2_ref, w3_ref, bb_ref, bbi_ref, bbo_ref):
    D = ws_ref.shape[0]
    wc1 = wc_ref[0:D, :]
    wc2 = wc_ref[D:2 * D, :]
    wc3 = wc_ref[2 * D:3 * D, :]
    w1_ref[...] = jnp.dot(ws_ref[...], wc1, preferred_element_type=jnp.float32)
    w2_ref[...] = jnp.dot(wi_ref[...], wc2, preferred_element_type=jnp.float32)
    w3_ref[...] = jnp.dot(wo_ref[...], wc3, preferred_element_type=jnp.float32)
    bb_ref[...] = jnp.dot(bs_ref[...], wc1, preferred_element_type=jnp.float32) + bc_ref[...]
    bbi_ref[...] = jnp.dot(bi_ref[...], wc2, preferred_element_type=jnp.float32)
    bbo_ref[...] = jnp.dot(bo_ref[...], wc3, preferred_element_type=jnp.float32)


def _final_body(x_ref, sin_ref, sout_ref, cin_ref, cout_ref,
                w1_ref, w2_ref, w3_ref, bb_ref, bbi_ref, bbo_ref, out_ref):
    ci = cin_ref[...]
    co = cout_ref[...]
    inv_i = 1.0 / jnp.maximum(ci, 1.0)
    inv_o = 1.0 / jnp.maximum(co, 1.0)
    ain = jnp.concatenate([sin_ref[0], sin_ref[1]], axis=-1) * inv_i
    aout = jnp.concatenate([sout_ref[0], sout_ref[1]], axis=-1) * inv_o
    acc = jnp.dot(x_ref[...], w1_ref[...], preferred_element_type=jnp.float32)
    acc += jnp.dot(ain, w2_ref[...], preferred_element_type=jnp.float32)
    acc += jnp.dot(aout, w3_ref[...], preferred_element_type=jnp.float32)
    acc += bb_ref[...]
    acc += jnp.where(ci > 0.0, 1.0, 0.0) * bbi_ref[...]
    acc += jnp.where(co > 0.0, 1.0, 0.0) * bbo_ref[...]
    out_ref[...] = jnp.maximum(acc, 0.0)


def kernel(x, senders, receivers, W_self, b_self, W_in, b_in, W_out, b_out,
           W_comb, b_comb):
    N, D = x.shape
    E = senders.shape[0]
    DH = D // 2

    TPT = -(-E // NS // EPW) * EPW
    E_pad = NS * TPT
    PAD = E_pad - E
    NROWS_SP = -(-(N + 1) // NS) * NS

    g_pad = jnp.arange(PAD, dtype=jnp.int32) % N
    s_pad = jnp.full((PAD,), N, dtype=jnp.int32)
    g_flat = jnp.concatenate([senders, g_pad, receivers, g_pad])
    s_flat = jnp.concatenate([receivers, s_pad, senders, s_pad])
    i_all = jnp.stack([g_flat.reshape(-1, EPW), s_flat.reshape(-1, EPW)],
                      axis=1)
    x0 = x[:, :DH]
    x1 = x[:, DH:]
    z2d = jnp.zeros((NROWS_SP // NS, DH), jnp.float32)
    z1d = jnp.zeros((NROWS_SP,), jnp.float32)

    agg = _sc_aggregate(E_pad, N, DH, TPT, NROWS_SP)
    sin_h, sout_h, cin_pad, cout_pad = agg(i_all, x0, x1, z2d, z1d)
    cin = cin_pad[:N]
    cout = cout_pad[:N]

    w1, w2, w3, bb, bbi, bbo = pl.pallas_call(
        _merge_body,
        out_shape=[
            jax.ShapeDtypeStruct((D, D), jnp.float32),
            jax.ShapeDtypeStruct((D, D), jnp.float32),
            jax.ShapeDtypeStruct((D, D), jnp.float32),
            jax.ShapeDtypeStruct((1, D), jnp.float32),
            jax.ShapeDtypeStruct((1, D), jnp.float32),
            jax.ShapeDtypeStruct((1, D), jnp.float32),
        ],
    )(W_self, W_in, W_out, W_comb,
      b_self.reshape(1, D), b_in.reshape(1, D), b_out.reshape(1, D),
      b_comb.reshape(1, D))

    BM = 2000
    grid = N // BM
    out = pl.pallas_call(
        _final_body,
        grid=(grid,),
        in_specs=[
            pl.BlockSpec((BM, D), lambda i: (i, 0)),
            pl.BlockSpec((2, BM, DH), lambda i: (0, i, 0)),
            pl.BlockSpec((2, BM, DH), lambda i: (0, i, 0)),
            pl.BlockSpec((BM, 1), lambda i: (i, 0)),
            pl.BlockSpec((BM, 1), lambda i: (i, 0)),
            pl.BlockSpec((D, D), lambda i: (0, 0)),
            pl.BlockSpec((D, D), lambda i: (0, 0)),
            pl.BlockSpec((D, D), lambda i: (0, 0)),
            pl.BlockSpec((1, D), lambda i: (0, 0)),
            pl.BlockSpec((1, D), lambda i: (0, 0)),
            pl.BlockSpec((1, D), lambda i: (0, 0)),
        ],
        out_specs=pl.BlockSpec((BM, D), lambda i: (i, 0)),
        out_shape=jax.ShapeDtypeStruct((N, D), jnp.float32),
    )(x, sin_h, sout_h, cin.reshape(N, 1), cout.reshape(N, 1),
      w1, w2, w3, bb, bbi, bbo)
    return out


# speedup vs baseline: 1.0407x; 1.0027x over previous
import functools
import jax
import jax.numpy as jnp
from jax import lax
from jax.experimental import pallas as pl
from jax.experimental.pallas import tpu as pltpu
from jax.experimental.pallas import tpu_sc as plsc

NC = 2
NS = 16
LANES = 16

EPW = 128


def _sc_aggregate(E_pad, N, DH, TPT, NROWS_SP, with_counts):
    NWIN = TPT // EPW
    ZROWS = NROWS_SP // NS
    WOUT = 640
    LAST = N - (NS - 1) * WOUT

    mesh = plsc.VectorSubcoreMesh(core_axis_name="c", subcore_axis_name="s")

    out_type = [
        jax.ShapeDtypeStruct((N, DH), jnp.float32),
        jax.ShapeDtypeStruct((N, DH), jnp.float32),
    ]
    if with_counts:
        out_type += [
            jax.ShapeDtypeStruct((NROWS_SP,), jnp.float32),
            jax.ShapeDtypeStruct((NROWS_SP,), jnp.float32),
        ]

    @functools.partial(
        pl.kernel,
        out_type=out_type,
        mesh=mesh,
        scratch_types=[
            pltpu.VMEM((3, 2, EPW), jnp.int32),
            pltpu.VMEM((3, EPW, DH), jnp.float32),
            pltpu.VMEM((EPW,), jnp.float32),
            pltpu.MemorySpace.VMEM_SHARED((NROWS_SP, DH), jnp.float32),
            pltpu.MemorySpace.VMEM_SHARED((NROWS_SP,), jnp.float32),
            pltpu.SemaphoreType.DMA((3,)),
            pltpu.SemaphoreType.DMA((3,)),
            pltpu.SemaphoreType.DMA((3,)),
            pltpu.SemaphoreType.DMA((3,)),
        ],
    )
    def agg(i_hbm, xh_hbm, z2d_hbm, z1d_hbm, *out_and_scratch):
        if with_counts:
            (sin_h, sout_h, cin, cout,
             ibuf, rows, ones, acc_sp, cnt_sp, si, sg, ss, sc) = out_and_scratch
        else:
            (sin_h, sout_h,
             ibuf, rows, ones, acc_sp, cnt_sp, si, sg, ss, sc) = out_and_scratch
        c = lax.axis_index("c")
        s = lax.axis_index("s")

        if with_counts:
            def fill_ones(i, _):
                ones[pl.ds(i * LANES, LANES)] = jnp.ones((LANES,), jnp.float32)
                return 0
            lax.fori_loop(0, EPW // LANES, fill_ones, 0)

        wbase = (c * NS + s) * NWIN

        pltpu.make_async_copy(i_hbm.at[wbase], ibuf.at[0], si.at[0]).start()

        pltpu.sync_copy(z2d_hbm, acc_sp.at[pl.ds(s * ZROWS, ZROWS), :])
        if with_counts:
            @pl.when(s == 0)
            def _():
                pltpu.sync_copy(z1d_hbm, cnt_sp)
        plsc.subcore_barrier()

        pltpu.make_async_copy(i_hbm.at[wbase + 1], ibuf.at[1],
                              si.at[1]).start()
        pltpu.make_async_copy(i_hbm.at[wbase], ibuf.at[0],
                              si.at[0]).wait()
        pltpu.async_copy(xh_hbm.at[ibuf.at[0, 0]], rows.at[0], sg.at[0])

        def step(w, _):
            s0 = lax.rem(w, 3)
            s1 = lax.rem(w + 1, 3)
            s2 = lax.rem(w + 2, 3)

            @pl.when(w + 1 < NWIN)
            def _():
                pltpu.make_async_copy(i_hbm.at[wbase + w + 1], ibuf.at[s1],
                                      si.at[s1]).wait()
                pltpu.async_copy(xh_hbm.at[ibuf.at[s1, 0]], rows.at[s1],
                                 sg.at[s1])

            pltpu.make_async_copy(xh_hbm.at[ibuf.at[s0, 0]], rows.at[s0],
                                  sg.at[s0]).wait()
            pltpu.async_copy(rows.at[s0], acc_sp.at[ibuf.at[s0, 1]],
                             ss.at[s0], add=True)
            if with_counts:
                pltpu.async_copy(ones, cnt_sp.at[ibuf.at[s0, 1]],
                                 sc.at[s0], add=True)

            @pl.when(w + 2 < NWIN)
            def _():
                @pl.when(w >= 1)
                def _():
                    pltpu.make_async_copy(rows.at[s2],
                                          acc_sp.at[ibuf.at[s2, 1]],
                                          ss.at[s2]).wait()
                    if with_counts:
                        pltpu.make_async_copy(ones,
                                              cnt_sp.at[ibuf.at[s2, 1]],
                                              sc.at[s2]).wait()
                pltpu.make_async_copy(i_hbm.at[wbase + w + 2], ibuf.at[s2],
                                      si.at[s2]).start()
            return 0
        lax.fori_loop(0, NWIN, step, 0)

        for wl in range(NWIN - 3, NWIN):
            pq = wl % 3
            pltpu.make_async_copy(rows.at[pq], acc_sp.at[ibuf.at[pq, 1]],
                                  ss.at[pq]).wait()
            if with_counts:
                pltpu.make_async_copy(ones, cnt_sp.at[ibuf.at[pq, 1]],
                                      sc.at[pq]).wait()
        plsc.subcore_barrier()

        for ci_, dst in ((0, sin_h), (1, sout_h)):
            @pl.when((c == ci_) & (s < NS - 1))
            def _(dst=dst):
                pltpu.sync_copy(acc_sp.at[pl.ds(s * WOUT, WOUT), :],
                                dst.at[pl.ds(s * WOUT, WOUT), :])

            @pl.when((c == ci_) & (s == NS - 1))
            def _(dst=dst):
                pltpu.sync_copy(acc_sp.at[pl.ds((NS - 1) * WOUT, LAST), :],
                                dst.at[pl.ds((NS - 1) * WOUT, LAST), :])

        if with_counts:
            @pl.when((c == 0) & (s == 0))
            def _():
                pltpu.sync_copy(cnt_sp, cin)

            @pl.when((c == 1) & (s == 0))
            def _():
                pltpu.sync_copy(cnt_sp, cout)

    return agg


def _merge_body(ws_ref, wi_ref, wo_ref, wc_ref, bs_ref, bi_ref, bo_ref, bc_ref,
                w1_ref, w2_ref, w3_ref, bb_ref, bbi_ref, bbo_ref):
    D = ws_ref.shape[0]
    wc1 = wc_ref[0:D, :]
    wc2 = wc_ref[D:2 * D, :]
    wc3 = wc_ref[2 * D:3 * D, :]
    w1_ref[...] = jnp.dot(ws_ref[...], wc1, preferred_element_type=jnp.float32)
    w2_ref[...] = jnp.dot(wi_ref[...], wc2, preferred_element_type=jnp.float32)
    w3_ref[...] = jnp.dot(wo_ref[...], wc3, preferred_element_type=jnp.float32)
    bb_ref[...] = jnp.dot(bs_ref[...], wc1, preferred_element_type=jnp.float32) + bc_ref[...]
    bbi_ref[...] = jnp.dot(bi_ref[...], wc2, preferred_element_type=jnp.float32)
    bbo_ref[...] = jnp.dot(bo_ref[...], wc3, preferred_element_type=jnp.float32)


def _ep1_body(x_ref, sin_ref, sout_ref, cin_ref, cout_ref,
              w1_ref, w2_ref, w3_ref, bb_ref, bbi_ref, bbo_ref, out_ref):
    DH = sin_ref.shape[-1]
    ci = cin_ref[...]
    co = cout_ref[...]
    inv_i = 1.0 / jnp.maximum(ci, 1.0)
    inv_o = 1.0 / jnp.maximum(co, 1.0)
    acc = jnp.dot(x_ref[...], w1_ref[...], preferred_element_type=jnp.float32)
    acc += jnp.dot(sin_ref[...] * inv_i, w2_ref[0:DH, :],
                   preferred_element_type=jnp.float32)
    acc += jnp.dot(sout_ref[...] * inv_o, w3_ref[0:DH, :],
                   preferred_element_type=jnp.float32)
    acc += bb_ref[...]
    acc += jnp.where(ci > 0.0, 1.0, 0.0) * bbi_ref[...]
    acc += jnp.where(co > 0.0, 1.0, 0.0) * bbo_ref[...]
    out_ref[...] = acc


def _ep2_body(acc_ref, sin_ref, sout_ref, cin_ref, cout_ref,
              w2_ref, w3_ref, out_ref):
    DH = sin_ref.shape[-1]
    inv_i = 1.0 / jnp.maximum(cin_ref[...], 1.0)
    inv_o = 1.0 / jnp.maximum(cout_ref[...], 1.0)
    acc = acc_ref[...]
    acc += jnp.dot(sin_ref[...] * inv_i, w2_ref[DH:2 * DH, :],
                   preferred_element_type=jnp.float32)
    acc += jnp.dot(sout_ref[...] * inv_o, w3_ref[DH:2 * DH, :],
                   preferred_element_type=jnp.float32)
    out_ref[...] = jnp.maximum(acc, 0.0)


def kernel(x, senders, receivers, W_self, b_self, W_in, b_in, W_out, b_out,
           W_comb, b_comb):
    N, D = x.shape
    E = senders.shape[0]
    DH = D // 2

    TPT = -(-E // NS // EPW) * EPW
    E_pad = NS * TPT
    PAD = E_pad - E
    NROWS_SP = -(-(N + 1) // NS) * NS

    g_pad = jnp.arange(PAD, dtype=jnp.int32) % N
    s_pad = jnp.full((PAD,), N, dtype=jnp.int32)
    g_flat = jnp.concatenate([senders, g_pad, receivers, g_pad])
    s_flat = jnp.concatenate([receivers, s_pad, senders, s_pad])
    i_all = jnp.stack([g_flat.reshape(-1, EPW), s_flat.reshape(-1, EPW)],
                      axis=1)
    x0 = x[:, :DH]
    x1 = x[:, DH:]
    z2d = jnp.zeros((NROWS_SP // NS, DH), jnp.float32)
    z1d = jnp.zeros((NROWS_SP,), jnp.float32)

    agg0 = _sc_aggregate(E_pad, N, DH, TPT, NROWS_SP, True)
    agg1 = _sc_aggregate(E_pad, N, DH, TPT, NROWS_SP, False)
    sin0, sout0, cin_pad, cout_pad = agg0(i_all, x0, z2d, z1d)
    sin1, sout1 = agg1(i_all, x1, z2d, z1d)
    cin = cin_pad[:N].reshape(N, 1)
    cout = cout_pad[:N].reshape(N, 1)

    w1, w2, w3, bb, bbi, bbo = pl.pallas_call(
        _merge_body,
        out_shape=[
            jax.ShapeDtypeStruct((D, D), jnp.float32),
            jax.ShapeDtypeStruct((D, D), jnp.float32),
            jax.ShapeDtypeStruct((D, D), jnp.float32),
            jax.ShapeDtypeStruct((1, D), jnp.float32),
            jax.ShapeDtypeStruct((1, D), jnp.float32),
            jax.ShapeDtypeStruct((1, D), jnp.float32),
        ],
    )(W_self, W_in, W_out, W_comb,
      b_self.reshape(1, D), b_in.reshape(1, D), b_out.reshape(1, D),
      b_comb.reshape(1, D))

    BM = 2000
    grid = N // BM
    full = lambda i: (0, 0)
    row = lambda i: (i, 0)
    acc = pl.pallas_call(
        _ep1_body,
        grid=(grid,),
        in_specs=[
            pl.BlockSpec((BM, D), row),
            pl.BlockSpec((BM, DH), row),
            pl.BlockSpec((BM, DH), row),
            pl.BlockSpec((BM, 1), row),
            pl.BlockSpec((BM, 1), row),
            pl.BlockSpec((D, D), full),
            pl.BlockSpec((D, D), full),
            pl.BlockSpec((D, D), full),
            pl.BlockSpec((1, D), full),
            pl.BlockSpec((1, D), full),
            pl.BlockSpec((1, D), full),
        ],
        out_specs=pl.BlockSpec((BM, D), row),
        out_shape=jax.ShapeDtypeStruct((N, D), jnp.float32),
    )(x, sin0, sout0, cin, cout, w1, w2, w3, bb, bbi, bbo)

    out = pl.pallas_call(
        _ep2_body,
        grid=(grid,),
        in_specs=[
            pl.BlockSpec((BM, D), row),
            pl.BlockSpec((BM, DH), row),
            pl.BlockSpec((BM, DH), row),
            pl.BlockSpec((BM, 1), row),
            pl.BlockSpec((BM, 1), row),
            pl.BlockSpec((D, D), full),
            pl.BlockSpec((D, D), full),
        ],
        out_specs=pl.BlockSpec((BM, D), row),
        out_shape=jax.ShapeDtypeStruct((N, D), jnp.float32),
    )(acc, sin1, sout1, cin, cout, w2, w3)
    return out


# speedup vs baseline: 1.0414x; 1.0007x over previous
import functools
import jax
import jax.numpy as jnp
from jax import lax
from jax.experimental import pallas as pl
from jax.experimental.pallas import tpu as pltpu
from jax.experimental.pallas import tpu_sc as plsc

NC = 2
NS = 16
LANES = 16

EPW = 128


def _sc_aggregate(E_pad, N, DH, TPT, NROWS_SP, with_counts):
    NWIN = TPT // EPW
    ZROWS = NROWS_SP // NS
    WOUT = 640
    LAST = N - (NS - 1) * WOUT

    mesh = plsc.VectorSubcoreMesh(core_axis_name="c", subcore_axis_name="s")

    out_type = [
        jax.ShapeDtypeStruct((N, DH), jnp.float32),
        jax.ShapeDtypeStruct((N, DH), jnp.float32),
    ]
    if with_counts:
        out_type += [
            jax.ShapeDtypeStruct((NROWS_SP,), jnp.float32),
            jax.ShapeDtypeStruct((NROWS_SP,), jnp.float32),
        ]

    @functools.partial(
        pl.kernel,
        out_type=out_type,
        mesh=mesh,
        scratch_types=[
            pltpu.VMEM((3, 2, EPW), jnp.int32),
            pltpu.VMEM((3, EPW, DH), jnp.float32),
            pltpu.VMEM((EPW,), jnp.float32),
            pltpu.MemorySpace.VMEM_SHARED((NROWS_SP, DH), jnp.float32),
            pltpu.MemorySpace.VMEM_SHARED((NROWS_SP,), jnp.float32),
            pltpu.SemaphoreType.DMA((3,)),
            pltpu.SemaphoreType.DMA((3,)),
            pltpu.SemaphoreType.DMA((3,)),
            pltpu.SemaphoreType.DMA((3,)),
        ],
    )
    def agg(i_hbm, xh_hbm, z2d_hbm, z1d_hbm, *out_and_scratch):
        if with_counts:
            (sin_h, sout_h, cin, cout,
             ibuf, rows, ones, acc_sp, cnt_sp, si, sg, ss, sc) = out_and_scratch
        else:
            (sin_h, sout_h,
             ibuf, rows, ones, acc_sp, cnt_sp, si, sg, ss, sc) = out_and_scratch
        c = lax.axis_index("c")
        s = lax.axis_index("s")

        if with_counts:
            def fill_ones(i, _):
                ones[pl.ds(i * LANES, LANES)] = jnp.ones((LANES,), jnp.float32)
                return 0
            lax.fori_loop(0, EPW // LANES, fill_ones, 0)

        wbase = (c * NS + s) * NWIN

        pltpu.make_async_copy(i_hbm.at[wbase], ibuf.at[0], si.at[0]).start()

        pltpu.sync_copy(z2d_hbm, acc_sp.at[pl.ds(s * ZROWS, ZROWS), :])
        if with_counts:
            @pl.when(s == 0)
            def _():
                pltpu.sync_copy(z1d_hbm, cnt_sp)
        plsc.subcore_barrier()

        def g_start(slot):
            for h in range(2):
                pltpu.async_copy(
                    xh_hbm.at[ibuf.at[slot, 0, pl.ds(h * 64, 64)]],
                    rows.at[slot, pl.ds(h * 64, 64), :], sg.at[slot])

        def g_wait(slot):
            for h in range(2):
                pltpu.make_async_copy(
                    xh_hbm.at[ibuf.at[slot, 0, pl.ds(h * 64, 64)]],
                    rows.at[slot, pl.ds(h * 64, 64), :], sg.at[slot]).wait()

        pltpu.make_async_copy(i_hbm.at[wbase + 1], ibuf.at[1],
                              si.at[1]).start()
        pltpu.make_async_copy(i_hbm.at[wbase], ibuf.at[0],
                              si.at[0]).wait()
        g_start(0)

        def step(w, _):
            s0 = lax.rem(w, 3)
            s1 = lax.rem(w + 1, 3)
            s2 = lax.rem(w + 2, 3)

            @pl.when(w + 1 < NWIN)
            def _():
                pltpu.make_async_copy(i_hbm.at[wbase + w + 1], ibuf.at[s1],
                                      si.at[s1]).wait()
                g_start(s1)

            g_wait(s0)
            pltpu.async_copy(rows.at[s0], acc_sp.at[ibuf.at[s0, 1]],
                             ss.at[s0], add=True)
            if with_counts:
                pltpu.async_copy(ones, cnt_sp.at[ibuf.at[s0, 1]],
                                 sc.at[s0], add=True)

            @pl.when(w + 2 < NWIN)
            def _():
                @pl.when(w >= 1)
                def _():
                    pltpu.make_async_copy(rows.at[s2],
                                          acc_sp.at[ibuf.at[s2, 1]],
                                          ss.at[s2]).wait()
                    if with_counts:
                        pltpu.make_async_copy(ones,
                                              cnt_sp.at[ibuf.at[s2, 1]],
                                              sc.at[s2]).wait()
                pltpu.make_async_copy(i_hbm.at[wbase + w + 2], ibuf.at[s2],
                                      si.at[s2]).start()
            return 0
        lax.fori_loop(0, NWIN, step, 0)

        for wl in range(NWIN - 3, NWIN):
            pq = wl % 3
            pltpu.make_async_copy(rows.at[pq], acc_sp.at[ibuf.at[pq, 1]],
                                  ss.at[pq]).wait()
            if with_counts:
                pltpu.make_async_copy(ones, cnt_sp.at[ibuf.at[pq, 1]],
                                      sc.at[pq]).wait()
        plsc.subcore_barrier()

        for ci_, dst in ((0, sin_h), (1, sout_h)):
            @pl.when((c == ci_) & (s < NS - 1))
            def _(dst=dst):
                pltpu.sync_copy(acc_sp.at[pl.ds(s * WOUT, WOUT), :],
                                dst.at[pl.ds(s * WOUT, WOUT), :])

            @pl.when((c == ci_) & (s == NS - 1))
            def _(dst=dst):
                pltpu.sync_copy(acc_sp.at[pl.ds((NS - 1) * WOUT, LAST), :],
                                dst.at[pl.ds((NS - 1) * WOUT, LAST), :])

        if with_counts:
            @pl.when((c == 0) & (s == 0))
            def _():
                pltpu.sync_copy(cnt_sp, cin)

            @pl.when((c == 1) & (s == 0))
            def _():
                pltpu.sync_copy(cnt_sp, cout)

    return agg


def _merge_body(ws_ref, wi_ref, wo_ref, wc_ref, bs_ref, bi_ref, bo_ref, bc_ref,
                w1_ref, w2_ref, w3_ref, bb_ref, bbi_ref, bbo_ref):
    D = ws_ref.shape[0]
    wc1 = wc_ref[0:D, :]
    wc2 = wc_ref[D:2 * D, :]
    wc3 = wc_ref[2 * D:3 * D, :]
    w1_ref[...] = jnp.dot(ws_ref[...], wc1, preferred_element_type=jnp.float32)
    w2_ref[...] = jnp.dot(wi_ref[...], wc2, preferred_element_type=jnp.float32)
    w3_ref[...] = jnp.dot(wo_ref[...], wc3, preferred_element_type=jnp.float32)
    bb_ref[...] = jnp.dot(bs_ref[...], wc1, preferred_element_type=jnp.float32) + bc_ref[...]
    bbi_ref[...] = jnp.dot(bi_ref[...], wc2, preferred_element_type=jnp.float32)
    bbo_ref[...] = jnp.dot(bo_ref[...], wc3, preferred_element_type=jnp.float32)


def _ep1_body(x_ref, sin_ref, sout_ref, cin_ref, cout_ref,
              w1_ref, w2_ref, w3_ref, bb_ref, bbi_ref, bbo_ref, out_ref):
    DH = sin_ref.shape[-1]
    ci = cin_ref[...]
    co = cout_ref[...]
    inv_i = 1.0 / jnp.maximum(ci, 1.0)
    inv_o = 1.0 / jnp.maximum(co, 1.0)
    acc = jnp.dot(x_ref[...], w1_ref[...], preferred_element_type=jnp.float32)
    acc += jnp.dot(sin_ref[...] * inv_i, w2_ref[0:DH, :],
                   preferred_element_type=jnp.float32)
    acc += jnp.dot(sout_ref[...] * inv_o, w3_ref[0:DH, :],
                   preferred_element_type=jnp.float32)
    acc += bb_ref[...]
    acc += jnp.where(ci > 0.0, 1.0, 0.0) * bbi_ref[...]
    acc += jnp.where(co > 0.0, 1.0, 0.0) * bbo_ref[...]
    out_ref[...] = acc


def _ep2_body(acc_ref, sin_ref, sout_ref, cin_ref, cout_ref,
              w2_ref, w3_ref, out_ref):
    DH = sin_ref.shape[-1]
    inv_i = 1.0 / jnp.maximum(cin_ref[...], 1.0)
    inv_o = 1.0 / jnp.maximum(cout_ref[...], 1.0)
    acc = acc_ref[...]
    acc += jnp.dot(sin_ref[...] * inv_i, w2_ref[DH:2 * DH, :],
                   preferred_element_type=jnp.float32)
    acc += jnp.dot(sout_ref[...] * inv_o, w3_ref[DH:2 * DH, :],
                   preferred_element_type=jnp.float32)
    out_ref[...] = jnp.maximum(acc, 0.0)


def kernel(x, senders, receivers, W_self, b_self, W_in, b_in, W_out, b_out,
           W_comb, b_comb):
    N, D = x.shape
    E = senders.shape[0]
    DH = D // 2

    TPT = -(-E // NS // EPW) * EPW
    E_pad = NS * TPT
    PAD = E_pad - E
    NROWS_SP = -(-(N + 1) // NS) * NS

    g_pad = jnp.arange(PAD, dtype=jnp.int32) % N
    s_pad = jnp.full((PAD,), N, dtype=jnp.int32)
    g_flat = jnp.concatenate([senders, g_pad, receivers, g_pad])
    s_flat = jnp.concatenate([receivers, s_pad, senders, s_pad])
    i_all = jnp.stack([g_flat.reshape(-1, EPW), s_flat.reshape(-1, EPW)],
                      axis=1)
    x0 = x[:, :DH]
    x1 = x[:, DH:]
    z2d = jnp.zeros((NROWS_SP // NS, DH), jnp.float32)
    z1d = jnp.zeros((NROWS_SP,), jnp.float32)

    agg0 = _sc_aggregate(E_pad, N, DH, TPT, NROWS_SP, True)
    agg1 = _sc_aggregate(E_pad, N, DH, TPT, NROWS_SP, False)
    sin0, sout0, cin_pad, cout_pad = agg0(i_all, x0, z2d, z1d)
    sin1, sout1 = agg1(i_all, x1, z2d, z1d)
    cin = cin_pad[:N].reshape(N, 1)
    cout = cout_pad[:N].reshape(N, 1)

    w1, w2, w3, bb, bbi, bbo = pl.pallas_call(
        _merge_body,
        out_shape=[
            jax.ShapeDtypeStruct((D, D), jnp.float32),
            jax.ShapeDtypeStruct((D, D), jnp.float32),
            jax.ShapeDtypeStruct((D, D), jnp.float32),
            jax.ShapeDtypeStruct((1, D), jnp.float32),
            jax.ShapeDtypeStruct((1, D), jnp.float32),
            jax.ShapeDtypeStruct((1, D), jnp.float32),
        ],
    )(W_self, W_in, W_out, W_comb,
      b_self.reshape(1, D), b_in.reshape(1, D), b_out.reshape(1, D),
      b_comb.reshape(1, D))

    BM = 2000
    grid = N // BM
    full = lambda i: (0, 0)
    row = lambda i: (i, 0)
    acc = pl.pallas_call(
        _ep1_body,
        grid=(grid,),
        in_specs=[
            pl.BlockSpec((BM, D), row),
            pl.BlockSpec((BM, DH), row),
            pl.BlockSpec((BM, DH), row),
            pl.BlockSpec((BM, 1), row),
            pl.BlockSpec((BM, 1), row),
            pl.BlockSpec((D, D), full),
            pl.BlockSpec((D, D), full),
            pl.BlockSpec((D, D), full),
            pl.BlockSpec((1, D), full),
            pl.BlockSpec((1, D), full),
            pl.BlockSpec((1, D), full),
        ],
        out_specs=pl.BlockSpec((BM, D), row),
        out_shape=jax.ShapeDtypeStruct((N, D), jnp.float32),
    )(x, sin0, sout0, cin, cout, w1, w2, w3, bb, bbi, bbo)

    out = pl.pallas_call(
        _ep2_body,
        grid=(grid,),
        in_specs=[
            pl.BlockSpec((BM, D), row),
            pl.BlockSpec((BM, DH), row),
            pl.BlockSpec((BM, DH), row),
            pl.BlockSpec((BM, 1), row),
            pl.BlockSpec((BM, 1), row),
            pl.BlockSpec((D, D), full),
            pl.BlockSpec((D, D), full),
        ],
        out_specs=pl.BlockSpec((BM, D), row),
        out_shape=jax.ShapeDtypeStruct((N, D), jnp.float32),
    )(acc, sin1, sout1, cin, cout, w2, w3)
    return out


# speedup vs baseline: 1.0430x; 1.0015x over previous
import functools
import jax
import jax.numpy as jnp
from jax import lax
from jax.experimental import pallas as pl
from jax.experimental.pallas import tpu as pltpu
from jax.experimental.pallas import tpu_sc as plsc

NC = 2
NS = 16
LANES = 16

EPW = 128


def _sc_aggregate(E_pad, N, DH, TPT, NROWS_SP, with_counts):
    NWIN = TPT // EPW
    ZROWS = NROWS_SP // NS
    WOUT = 640
    LAST = N - (NS - 1) * WOUT

    mesh = plsc.VectorSubcoreMesh(core_axis_name="c", subcore_axis_name="s")

    out_type = [
        jax.ShapeDtypeStruct((N, DH), jnp.float32),
        jax.ShapeDtypeStruct((N, DH), jnp.float32),
    ]
    if with_counts:
        out_type += [
            jax.ShapeDtypeStruct((NROWS_SP,), jnp.float32),
            jax.ShapeDtypeStruct((NROWS_SP,), jnp.float32),
        ]

    @functools.partial(
        pl.kernel,
        out_type=out_type,
        mesh=mesh,
        scratch_types=[
            pltpu.VMEM((3, 2, EPW), jnp.int32),
            pltpu.VMEM((3, EPW, DH), jnp.float32),
            pltpu.VMEM((EPW,), jnp.float32),
            pltpu.MemorySpace.VMEM_SHARED((NROWS_SP, DH), jnp.float32),
            pltpu.MemorySpace.VMEM_SHARED((NROWS_SP,), jnp.float32),
            pltpu.SemaphoreType.DMA((3,)),
            pltpu.SemaphoreType.DMA((3,)),
            pltpu.SemaphoreType.DMA((3,)),
            pltpu.SemaphoreType.DMA((3,)),
        ],
    )
    def agg(i_hbm, xh_hbm, z2d_hbm, z1d_hbm, *out_and_scratch):
        if with_counts:
            (sin_h, sout_h, cin, cout,
             ibuf, rows, ones, acc_sp, cnt_sp, si, sg, ss, sc) = out_and_scratch
        else:
            (sin_h, sout_h,
             ibuf, rows, ones, acc_sp, cnt_sp, si, sg, ss, sc) = out_and_scratch
        c = lax.axis_index("c")
        s = lax.axis_index("s")

        if with_counts:
            def fill_ones(i, _):
                ones[pl.ds(i * LANES, LANES)] = jnp.ones((LANES,), jnp.float32)
                return 0
            lax.fori_loop(0, EPW // LANES, fill_ones, 0)

        wbase = (c * NS + s) * NWIN

        pltpu.make_async_copy(i_hbm.at[wbase], ibuf.at[0], si.at[0]).start()

        pltpu.sync_copy(z2d_hbm, acc_sp.at[pl.ds(s * ZROWS, ZROWS), :])
        if with_counts:
            @pl.when(s == 0)
            def _():
                pltpu.sync_copy(z1d_hbm, cnt_sp)
        plsc.subcore_barrier()

        def g_start(slot):
            for h in range(2):
                pltpu.async_copy(
                    xh_hbm.at[ibuf.at[slot, 0, pl.ds(h * 64, 64)]],
                    rows.at[slot, pl.ds(h * 64, 64), :], sg.at[slot])

        def g_wait(slot):
            for h in range(2):
                pltpu.make_async_copy(
                    xh_hbm.at[ibuf.at[slot, 0, pl.ds(h * 64, 64)]],
                    rows.at[slot, pl.ds(h * 64, 64), :], sg.at[slot]).wait()

        pltpu.make_async_copy(i_hbm.at[wbase + 1], ibuf.at[1],
                              si.at[1]).start()
        pltpu.make_async_copy(i_hbm.at[wbase], ibuf.at[0],
                              si.at[0]).wait()
        g_start(0)

        def step(w, _):
            s0 = lax.rem(w, 3)
            s1 = lax.rem(w + 1, 3)
            s2 = lax.rem(w + 2, 3)

            @pl.when(w + 1 < NWIN)
            def _():
                pltpu.make_async_copy(i_hbm.at[wbase + w + 1], ibuf.at[s1],
                                      si.at[s1]).wait()
                g_start(s1)

            g_wait(s0)
            pltpu.async_copy(rows.at[s0], acc_sp.at[ibuf.at[s0, 1]],
                             ss.at[s0], add=True)
            if with_counts:
                pltpu.async_copy(ones, cnt_sp.at[ibuf.at[s0, 1]],
                                 sc.at[s0], add=True)

            @pl.when(w + 2 < NWIN)
            def _():
                @pl.when(w >= 1)
                def _():
                    pltpu.make_async_copy(rows.at[s2],
                                          acc_sp.at[ibuf.at[s2, 1]],
                                          ss.at[s2]).wait()
                    if with_counts:
                        pltpu.make_async_copy(ones,
                                              cnt_sp.at[ibuf.at[s2, 1]],
                                              sc.at[s2]).wait()
                pltpu.make_async_copy(i_hbm.at[wbase + w + 2], ibuf.at[s2],
                                      si.at[s2]).start()
            return 0
        lax.fori_loop(0, NWIN, step, 0, unroll=2)

        for wl in range(NWIN - 3, NWIN):
            pq = wl % 3
            pltpu.make_async_copy(rows.at[pq], acc_sp.at[ibuf.at[pq, 1]],
                                  ss.at[pq]).wait()
            if with_counts:
                pltpu.make_async_copy(ones, cnt_sp.at[ibuf.at[pq, 1]],
                                      sc.at[pq]).wait()
        plsc.subcore_barrier()

        for ci_, dst in ((0, sin_h), (1, sout_h)):
            @pl.when((c == ci_) & (s < NS - 1))
            def _(dst=dst):
                pltpu.sync_copy(acc_sp.at[pl.ds(s * WOUT, WOUT), :],
                                dst.at[pl.ds(s * WOUT, WOUT), :])

            @pl.when((c == ci_) & (s == NS - 1))
            def _(dst=dst):
                pltpu.sync_copy(acc_sp.at[pl.ds((NS - 1) * WOUT, LAST), :],
                                dst.at[pl.ds((NS - 1) * WOUT, LAST), :])

        if with_counts:
            @pl.when((c == 0) & (s == 0))
            def _():
                pltpu.sync_copy(cnt_sp, cin)

            @pl.when((c == 1) & (s == 0))
            def _():
                pltpu.sync_copy(cnt_sp, cout)

    return agg


def _merge_body(ws_ref, wi_ref, wo_ref, wc_ref, bs_ref, bi_ref, bo_ref, bc_ref,
                w1_ref, w2_ref, w3_ref, bb_ref, bbi_ref, bbo_ref):
    D = ws_ref.shape[0]
    wc1 = wc_ref[0:D, :]
    wc2 = wc_ref[D:2 * D, :]
    wc3 = wc_ref[2 * D:3 * D, :]
    w1_ref[...] = jnp.dot(ws_ref[...], wc1, preferred_element_type=jnp.float32)
    w2_ref[...] = jnp.dot(wi_ref[...], wc2, preferred_element_type=jnp.float32)
    w3_ref[...] = jnp.dot(wo_ref[...], wc3, preferred_element_type=jnp.float32)
    bb_ref[...] = jnp.dot(bs_ref[...], wc1, preferred_element_type=jnp.float32) + bc_ref[...]
    bbi_ref[...] = jnp.dot(bi_ref[...], wc2, preferred_element_type=jnp.float32)
    bbo_ref[...] = jnp.dot(bo_ref[...], wc3, preferred_element_type=jnp.float32)


def _ep1_body(x_ref, sin_ref, sout_ref, cin_ref, cout_ref,
              w1_ref, w2_ref, w3_ref, bb_ref, bbi_ref, bbo_ref, out_ref):
    DH = sin_ref.shape[-1]
    ci = cin_ref[...]
    co = cout_ref[...]
    inv_i = 1.0 / jnp.maximum(ci, 1.0)
    inv_o = 1.0 / jnp.maximum(co, 1.0)
    acc = jnp.dot(x_ref[...], w1_ref[...], preferred_element_type=jnp.float32)
    acc += jnp.dot(sin_ref[...] * inv_i, w2_ref[0:DH, :],
                   preferred_element_type=jnp.float32)
    acc += jnp.dot(sout_ref[...] * inv_o, w3_ref[0:DH, :],
                   preferred_element_type=jnp.float32)
    acc += bb_ref[...]
    acc += jnp.where(ci > 0.0, 1.0, 0.0) * bbi_ref[...]
    acc += jnp.where(co > 0.0, 1.0, 0.0) * bbo_ref[...]
    out_ref[...] = acc


def _ep2_body(acc_ref, sin_ref, sout_ref, cin_ref, cout_ref,
              w2_ref, w3_ref, out_ref):
    DH = sin_ref.shape[-1]
    inv_i = 1.0 / jnp.maximum(cin_ref[...], 1.0)
    inv_o = 1.0 / jnp.maximum(cout_ref[...], 1.0)
    acc = acc_ref[...]
    acc += jnp.dot(sin_ref[...] * inv_i, w2_ref[DH:2 * DH, :],
                   preferred_element_type=jnp.float32)
    acc += jnp.dot(sout_ref[...] * inv_o, w3_ref[DH:2 * DH, :],
                   preferred_element_type=jnp.float32)
    out_ref[...] = jnp.maximum(acc, 0.0)


def kernel(x, senders, receivers, W_self, b_self, W_in, b_in, W_out, b_out,
           W_comb, b_comb):
    N, D = x.shape
    E = senders.shape[0]
    DH = D // 2

    TPT = -(-E // NS // EPW) * EPW
    E_pad = NS * TPT
    PAD = E_pad - E
    NROWS_SP = -(-(N + 1) // NS) * NS

    g_pad = jnp.arange(PAD, dtype=jnp.int32) % N
    s_pad = jnp.full((PAD,), N, dtype=jnp.int32)
    g_flat = jnp.concatenate([senders, g_pad, receivers, g_pad])
    s_flat = jnp.concatenate([receivers, s_pad, senders, s_pad])
    i_all = jnp.stack([g_flat.reshape(-1, EPW), s_flat.reshape(-1, EPW)],
                      axis=1)
    x0 = x[:, :DH]
    x1 = x[:, DH:]
    z2d = jnp.zeros((NROWS_SP // NS, DH), jnp.float32)
    z1d = jnp.zeros((NROWS_SP,), jnp.float32)

    agg0 = _sc_aggregate(E_pad, N, DH, TPT, NROWS_SP, True)
    agg1 = _sc_aggregate(E_pad, N, DH, TPT, NROWS_SP, False)
    sin0, sout0, cin_pad, cout_pad = agg0(i_all, x0, z2d, z1d)
    sin1, sout1 = agg1(i_all, x1, z2d, z1d)
    cin = cin_pad[:N].reshape(N, 1)
    cout = cout_pad[:N].reshape(N, 1)

    w1, w2, w3, bb, bbi, bbo = pl.pallas_call(
        _merge_body,
        out_shape=[
            jax.ShapeDtypeStruct((D, D), jnp.float32),
            jax.ShapeDtypeStruct((D, D), jnp.float32),
            jax.ShapeDtypeStruct((D, D), jnp.float32),
            jax.ShapeDtypeStruct((1, D), jnp.float32),
            jax.ShapeDtypeStruct((1, D), jnp.float32),
            jax.ShapeDtypeStruct((1, D), jnp.float32),
        ],
    )(W_self, W_in, W_out, W_comb,
      b_self.reshape(1, D), b_in.reshape(1, D), b_out.reshape(1, D),
      b_comb.reshape(1, D))

    BM = 2000
    grid = N // BM
    full = lambda i: (0, 0)
    row = lambda i: (i, 0)
    acc = pl.pallas_call(
        _ep1_body,
        grid=(grid,),
        in_specs=[
            pl.BlockSpec((BM, D), row),
            pl.BlockSpec((BM, DH), row),
            pl.BlockSpec((BM, DH), row),
            pl.BlockSpec((BM, 1), row),
            pl.BlockSpec((BM, 1), row),
            pl.BlockSpec((D, D), full),
            pl.BlockSpec((D, D), full),
            pl.BlockSpec((D, D), full),
            pl.BlockSpec((1, D), full),
            pl.BlockSpec((1, D), full),
            pl.BlockSpec((1, D), full),
        ],
        out_specs=pl.BlockSpec((BM, D), row),
        out_shape=jax.ShapeDtypeStruct((N, D), jnp.float32),
    )(x, sin0, sout0, cin, cout, w1, w2, w3, bb, bbi, bbo)

    out = pl.pallas_call(
        _ep2_body,
        grid=(grid,),
        in_specs=[
            pl.BlockSpec((BM, D), row),
            pl.BlockSpec((BM, DH), row),
            pl.BlockSpec((BM, DH), row),
            pl.BlockSpec((BM, 1), row),
            pl.BlockSpec((BM, 1), row),
            pl.BlockSpec((D, D), full),
            pl.BlockSpec((D, D), full),
        ],
        out_specs=pl.BlockSpec((BM, D), row),
        out_shape=jax.ShapeDtypeStruct((N, D), jnp.float32),
    )(acc, sin1, sout1, cin, cout, w2, w3)
    return out
